# Initial kernel scaffold; baseline (speedup 1.0000x reference)
import functools

import jax
import jax.numpy as jnp
from jax import lax
from jax.experimental import pallas as pl
from jax.experimental.pallas import tpu as pltpu
from jax.experimental.pallas import tpu_sc as plsc

_EMB_DIM = 64
_N_PAIRS = 16384
_K_NEG = 5
_NC = 2
_NS = 16
_NW = _NC * _NS
_PAIRS_PER_W = _N_PAIRS // _NW
_CHUNK = 128
_NCHUNK = _PAIRS_PER_W // _CHUNK
_GROUPS = _CHUNK // 16

_LN2 = 0.6931471805599453
_SQRT2 = 1.4142135


def _softplus(x):
    y = 1.0 + jnp.exp(x)
    bits = plsc.bitcast(y, jnp.int32)
    e = (bits >> 23) - 127
    m = plsc.bitcast((bits & 0x007FFFFF) | 0x3F800000, jnp.float32)
    adj = m > _SQRT2
    m = jnp.where(adj, m * 0.5, m)
    e = jnp.where(adj, e + 1, e)
    z = (m - 1.0) / (m + 1.0)
    z2 = z * z
    p = z * (2.0 + z2 * ((2.0 / 3.0) + z2 * ((2.0 / 5.0) + z2 * (2.0 / 7.0))))
    return e.astype(jnp.float32) * _LN2 + p


def _sc_body(u_hbm, v_hbm, pos_u_hbm, pos_v_hbm, neg2d_hbm, out_hbm,
             idx_u, idx_v, idx_n, u_buf, v_buf, n_buf, acc_vmem, sem):
    wid = lax.axis_index("s") * _NC + lax.axis_index("c")
    lanes = lax.iota(jnp.int32, 16)

    def chunk_body(c, loss):
        base = wid * _PAIRS_PER_W + c * _CHUNK
        pltpu.sync_copy(pos_u_hbm.at[pl.ds(base, _CHUNK)], idx_u)
        pltpu.sync_copy(pos_v_hbm.at[pl.ds(base, _CHUNK)], idx_v)
        nrow = wid * (_PAIRS_PER_W * _K_NEG // 128) + c * _K_NEG
        pltpu.sync_copy(neg2d_hbm.at[pl.ds(nrow, _K_NEG)], idx_n)
        cps = [pltpu.async_copy(u_hbm.at[idx_u], u_buf, sem),
               pltpu.async_copy(v_hbm.at[idx_v], v_buf, sem)]
        for j in range(_K_NEG):
            cps.append(pltpu.async_copy(
                v_hbm.at[idx_n.at[j]], n_buf.at[pl.ds(j * 128, 128)], sem))
        for cp in cps:
            cp.wait()

        def group_body(g, gloss):
            pairs = g * 16 + lanes
            nrows = [pairs * _K_NEG + j for j in range(_K_NEG)]
            acc = [jnp.zeros((16,), jnp.float32) for _ in range(1 + _K_NEG)]
            for d in range(_EMB_DIM):
                col = jnp.full((16,), d, jnp.int32)
                au = plsc.load_gather(u_buf, [pairs, col])
                av = plsc.load_gather(v_buf, [pairs, col])
                acc[0] = acc[0] + au * av
                for j in range(_K_NEG):
                    an = plsc.load_gather(n_buf, [nrows[j], col])
                    acc[1 + j] = acc[1 + j] + au * an
            s = jnp.clip(acc[0], -6.0, 6.0)
            gloss = gloss + _softplus(-s)
            for j in range(_K_NEG):
                sj = jnp.clip(acc[1 + j], -6.0, 6.0)
                gloss = gloss + _softplus(sj)
            return gloss

        return lax.fori_loop(0, _GROUPS, group_body, loss)

    loss = lax.fori_loop(0, _NCHUNK, chunk_body,
                         jnp.zeros((16,), jnp.float32))
    acc_vmem[...] = loss
    pltpu.sync_copy(acc_vmem, out_hbm.at[wid])


@jax.jit
def _sc_partials(u_weight, v_weight, pos_u, pos_v, neg2d):
    mesh = plsc.VectorSubcoreMesh(core_axis_name="c", subcore_axis_name="s")
    f = pl.kernel(
        _sc_body,
        out_type=jax.ShapeDtypeStruct((_NW, 16), jnp.float32),
        mesh=mesh,
        scratch_types=[
            pltpu.VMEM((_CHUNK,), jnp.int32),
            pltpu.VMEM((_CHUNK,), jnp.int32),
            pltpu.VMEM((_K_NEG, 128), jnp.int32),
            pltpu.VMEM((_CHUNK, _EMB_DIM), jnp.float32),
            pltpu.VMEM((_CHUNK, _EMB_DIM), jnp.float32),
            pltpu.VMEM((_CHUNK * _K_NEG, _EMB_DIM), jnp.float32),
            pltpu.VMEM((16,), jnp.float32),
            pltpu.SemaphoreType.DMA,
        ],
    )
    return f(u_weight, v_weight, pos_u, pos_v, neg2d)


def _reduce_body(x_ref, o_ref):
    o_ref[0, 0] = jnp.sum(x_ref[...]) * (1.0 / _N_PAIRS)


@jax.jit
def kernel(u_weight, v_weight, pos_u, pos_v, neg_v):
    neg2d = neg_v.reshape(_N_PAIRS * _K_NEG // 128, 128)
    partials = _sc_partials(u_weight, v_weight, pos_u, pos_v, neg2d)
    total = pl.pallas_call(
        _reduce_body,
        out_shape=jax.ShapeDtypeStruct((1, 1), jnp.float32),
        out_specs=pl.BlockSpec(memory_space=pltpu.SMEM),
    )(partials)
    return total[0, 0]


# speedup vs baseline: 1.5958x; 1.5958x over previous
import functools

import jax
import jax.numpy as jnp
from jax import lax
from jax.experimental import pallas as pl
from jax.experimental.pallas import tpu as pltpu
from jax.experimental.pallas import tpu_sc as plsc

_EMB_DIM = 64
_N_PAIRS = 16384
_K_NEG = 5
_NC = 2
_NS = 16
_NW = _NC * _NS
_PAIRS_PER_W = _N_PAIRS // _NW
_CHUNK = 128
_NCHUNK = _PAIRS_PER_W // _CHUNK
_GROUPS = _CHUNK // 16

_LN2 = 0.6931471805599453
_SQRT2 = 1.4142135


def _softplus(x):
    y = 1.0 + jnp.exp(x)
    bits = plsc.bitcast(y, jnp.int32)
    e = (bits >> 23) - 127
    m = plsc.bitcast((bits & 0x007FFFFF) | 0x3F800000, jnp.float32)
    adj = m > _SQRT2
    m = jnp.where(adj, m * 0.5, m)
    e = jnp.where(adj, e + 1, e)
    z = (m - 1.0) / (m + 1.0)
    z2 = z * z
    p = z * (2.0 + z2 * ((2.0 / 3.0) + z2 * ((2.0 / 5.0) + z2 * (2.0 / 7.0))))
    return e.astype(jnp.float32) * _LN2 + p


def _sc_body(u_hbm, v_hbm, pos_u_hbm, pos_v_hbm, neg_flat_hbm, out_hbm,
             idx_u, idx_v, idx_n, u_buf, v_buf, n_buf, acc_vmem, sem):
    wid = lax.axis_index("s") * _NC + lax.axis_index("c")
    lanes = lax.iota(jnp.int32, 16)

    def chunk_body(c, loss):
        base = wid * _PAIRS_PER_W + c * _CHUNK
        pltpu.sync_copy(pos_u_hbm.at[pl.ds(base, _CHUNK)], idx_u)
        pltpu.sync_copy(pos_v_hbm.at[pl.ds(base, _CHUNK)], idx_v)
        pltpu.sync_copy(neg_flat_hbm.at[pl.ds(base * _K_NEG,
                                              _CHUNK * _K_NEG)], idx_n)
        cps = [pltpu.async_copy(u_hbm.at[idx_u], u_buf, sem),
               pltpu.async_copy(v_hbm.at[idx_v], v_buf, sem)]
        for j in range(_K_NEG):
            cps.append(pltpu.async_copy(
                v_hbm.at[idx_n.at[pl.ds(j * 128, 128)]],
                n_buf.at[pl.ds(j * 128, 128)], sem))
        for cp in cps:
            cp.wait()

        def group_body(g, gloss):
            pairs = g * 16 + lanes
            nrows = [pairs * _K_NEG + j for j in range(_K_NEG)]
            acc = [jnp.zeros((16,), jnp.float32) for _ in range(1 + _K_NEG)]
            for d in range(_EMB_DIM):
                col = jnp.full((16,), d, jnp.int32)
                au = plsc.load_gather(u_buf, [pairs, col])
                av = plsc.load_gather(v_buf, [pairs, col])
                acc[0] = acc[0] + au * av
                for j in range(_K_NEG):
                    an = plsc.load_gather(n_buf, [nrows[j], col])
                    acc[1 + j] = acc[1 + j] + au * an
            s = jnp.clip(acc[0], -6.0, 6.0)
            gloss = gloss + _softplus(-s)
            for j in range(_K_NEG):
                sj = jnp.clip(acc[1 + j], -6.0, 6.0)
                gloss = gloss + _softplus(sj)
            return gloss

        return lax.fori_loop(0, _GROUPS, group_body, loss)

    loss = lax.fori_loop(0, _NCHUNK, chunk_body,
                         jnp.zeros((16,), jnp.float32))
    acc_vmem[...] = loss
    pltpu.sync_copy(acc_vmem, out_hbm.at[wid])


@jax.jit
def _sc_partials(u_weight, v_weight, pos_u, pos_v, neg_flat):
    mesh = plsc.VectorSubcoreMesh(core_axis_name="c", subcore_axis_name="s")
    f = pl.kernel(
        _sc_body,
        out_type=jax.ShapeDtypeStruct((_NW, 16), jnp.float32),
        mesh=mesh,
        compiler_params=pltpu.CompilerParams(needs_layout_passes=False,
                                             use_tc_tiling_on_sc=False),
        scratch_types=[
            pltpu.VMEM((_CHUNK,), jnp.int32),
            pltpu.VMEM((_CHUNK,), jnp.int32),
            pltpu.VMEM((_CHUNK * _K_NEG,), jnp.int32),
            pltpu.VMEM((_CHUNK, _EMB_DIM), jnp.float32),
            pltpu.VMEM((_CHUNK, _EMB_DIM), jnp.float32),
            pltpu.VMEM((_CHUNK * _K_NEG, _EMB_DIM), jnp.float32),
            pltpu.VMEM((16,), jnp.float32),
            pltpu.SemaphoreType.DMA,
        ],
    )
    return f(u_weight, v_weight, pos_u, pos_v, neg_flat)


def _reduce_body(x_ref, o_ref):
    o_ref[0, 0] = jnp.sum(x_ref[...]) * (1.0 / _N_PAIRS)


@jax.jit
def kernel(u_weight, v_weight, pos_u, pos_v, neg_v):
    neg_flat = neg_v.reshape(_N_PAIRS * _K_NEG)
    partials = _sc_partials(u_weight, v_weight, pos_u, pos_v, neg_flat)
    total = pl.pallas_call(
        _reduce_body,
        out_shape=jax.ShapeDtypeStruct((1, 1), jnp.float32),
        out_specs=pl.BlockSpec(memory_space=pltpu.SMEM),
    )(partials)
    return total[0, 0]
